# baseline (speedup 1.0000x reference)
import functools

import jax
import jax.numpy as jnp
from jax import lax
from jax.experimental import pallas as pl
from jax.experimental.pallas import tpu as pltpu
from jax.experimental.pallas import tpu_sc as plsc

N = 10000
E = 320000
C = 128
NC, NS = 2, 16
NW = NC * NS
EPW = E // NW
CHUNK = 80
NCHUNK = EPW // CHUNK
PASSES = 5
PCHUNK = NCHUNK // PASSES
PEDGE = PCHUNK * CHUNK
NPAD = 10240
DEG_SLAB = NPAD // NS
ROW_SLAB = 640
ROW_STRIDE = 624
RB = 2000
GRID = N // RB

_mesh = plsc.VectorSubcoreMesh(core_axis_name="c", subcore_axis_name="s")


def _zero_vmem_1d(ref, n):
    def body(i, _):
        ref[pl.ds(i * 16, 16)] = jnp.zeros((16,), jnp.float32)
        return 0
    lax.fori_loop(0, n // 16, body, 0)


def _deg_body(dst_hbm, w_hbm, deg_out, didx, wbuf, zbuf, acc, sem):
    c = lax.axis_index("c")
    s = lax.axis_index("s")
    wid = s * NC + c
    base = wid * EPW
    pltpu.async_copy(dst_hbm.at[pl.ds(base, EPW)], didx, sem)
    pltpu.async_copy(w_hbm.at[pl.ds(base, EPW)], wbuf, sem)
    _zero_vmem_1d(zbuf, DEG_SLAB)
    pltpu.sync_copy(zbuf, acc.at[pl.ds(s * DEG_SLAB, DEG_SLAB)])
    pltpu.make_async_copy(dst_hbm.at[pl.ds(base, EPW)], didx, sem).wait()
    pltpu.make_async_copy(w_hbm.at[pl.ds(base, EPW)], wbuf, sem).wait()
    plsc.subcore_barrier()

    def chunk(j, _):
        pltpu.sync_copy(wbuf.at[pl.ds(j * CHUNK, CHUNK)],
                        acc.at[didx.at[pl.ds(j * CHUNK, CHUNK)]], add=True)
        return 0

    lax.fori_loop(0, NCHUNK, chunk, 0)
    plsc.subcore_barrier()
    pltpu.sync_copy(acc.at[pl.ds(s * DEG_SLAB, DEG_SLAB)],
                    deg_out.at[c, pl.ds(s * DEG_SLAB, DEG_SLAB)])


_deg_call = pl.kernel(
    _deg_body,
    out_type=jax.ShapeDtypeStruct((NC, NPAD), jnp.float32),
    mesh=_mesh,
    scratch_types=[
        pltpu.VMEM((EPW,), jnp.int32),
        pltpu.VMEM((EPW,), jnp.float32),
        pltpu.VMEM((DEG_SLAB,), jnp.float32),
        pltpu.VMEM_SHARED((NPAD,), jnp.float32),
        pltpu.SemaphoreType.DMA,
    ],
)


def _scale_rows(rows, wbuf, j, q0=0, q1=CHUNK // 16):
    for q in range(q0, q1):
        wvec = wbuf[pl.ds(j * CHUNK + q * 16, 16)]

        def scale_row(r, _2, q=q, wvec=wvec):
            wspl = lax.gather(
                wvec, jnp.full((16, 1), r, jnp.int32),
                lax.GatherDimensionNumbers(
                    offset_dims=(), collapsed_slice_dims=(0,),
                    start_index_map=(0,)),
                (1,), mode=lax.GatherScatterMode.PROMISE_IN_BOUNDS)
            i = q * 16 + r
            for k in range(C // 16):
                rows[i, pl.ds(k * 16, 16)] = rows[i, pl.ds(k * 16, 16)] * wspl
            return 0

        lax.fori_loop(0, 16, scale_row, 0, unroll=4)


def _spmm_body(src_hbm, dst_hbm, w_hbm, g_hbm, out_hbm,
               sidx, didx, wbuf, rows0, rows1, rows2, acc,
               sems, semg0, semg1, semg2, semsc0, semsc1, semsc2):
    c = lax.axis_index("c")
    s = lax.axis_index("s")
    wid = s * NC + c
    rows = (rows0, rows1, rows2)
    semg = (semg0, semg1, semg2)
    semsc = (semsc0, semsc1, semsc2)

    def stage(p_):
        base = wid * EPW + p_ * PEDGE
        pltpu.async_copy(src_hbm.at[pl.ds(base, PEDGE)], sidx, sems)
        pltpu.async_copy(dst_hbm.at[pl.ds(base, PEDGE)], didx, sems)
        pltpu.async_copy(w_hbm.at[pl.ds(base, PEDGE)], wbuf, sems)

    def stage_wait(p_):
        base = wid * EPW + p_ * PEDGE
        pltpu.make_async_copy(src_hbm.at[pl.ds(base, PEDGE)], sidx, sems).wait()
        pltpu.make_async_copy(dst_hbm.at[pl.ds(base, PEDGE)], didx, sems).wait()
        pltpu.make_async_copy(w_hbm.at[pl.ds(base, PEDGE)], wbuf, sems).wait()

    stage(0)
    def zrow(i, _):
        def zcol(k, _2):
            rows0[i, pl.ds(k * 16, 16)] = jnp.zeros((16,), jnp.float32)
            return 0
        lax.fori_loop(0, C // 16, zcol, 0)
        return 0
    lax.fori_loop(0, CHUNK, zrow, 0)
    r0 = s * ROW_STRIDE
    for t in range(ROW_SLAB // CHUNK):
        pltpu.sync_copy(rows0, acc.at[pl.ds(r0 + t * CHUNK, CHUNK)])
    plsc.subcore_barrier()

    def gat(j, b):
        pltpu.async_copy(g_hbm.at[sidx.at[pl.ds(j * CHUNK, CHUNK)]],
                         rows[b], semg[b])

    def gat_wait(j, b):
        pltpu.make_async_copy(g_hbm.at[sidx.at[pl.ds(j * CHUNK, CHUNK)]],
                              rows[b], semg[b]).wait()

    H1 = 32

    def sca_part(j, b, r_off, nrows):
        pltpu.async_copy(rows[b].at[pl.ds(r_off, nrows)],
                         acc.at[didx.at[pl.ds(j * CHUNK + r_off, nrows)]],
                         semsc[b], add=True)

    def sca_wait(b):
        pltpu.make_async_copy(rows[b].at[pl.ds(0, H1)],
                              acc.at[didx.at[pl.ds(0, H1)]], semsc[b]).wait()
        pltpu.make_async_copy(rows[b].at[pl.ds(0, CHUNK - H1)],
                              acc.at[didx.at[pl.ds(0, CHUNK - H1)]],
                              semsc[b]).wait()

    def scale_scatter(j, b):
        _scale_rows(rows[b], wbuf, j, 0, H1 // 16)
        sca_part(j, b, 0, H1)
        _scale_rows(rows[b], wbuf, j, H1 // 16, CHUNK // 16)
        sca_part(j, b, H1, CHUNK - H1)

    def one_pass(p_, _):
        stage_wait(p_)
        gat(0, 0)
        gat(1, 1)
        gat_wait(0, 0)
        gat(2, 2)
        scale_scatter(0, 0)
        gat_wait(1, 1)
        sca_wait(0)
        gat(3, 0)
        scale_scatter(1, 1)

        def triple(t, _):
            for off, b, b2 in ((2, 2, 1), (3, 0, 2), (4, 1, 0)):
                j = 3 * t + off
                gat_wait(j, b)
                sca_wait(b2)
                gat(j + 2, b2)
                scale_scatter(j, b)
            return 0

        lax.fori_loop(0, (PCHUNK - 4) // 3, triple, 0)
        gat_wait(PCHUNK - 2, 2)
        sca_wait(1)
        scale_scatter(PCHUNK - 2, 2)
        gat_wait(PCHUNK - 1, 0)
        sca_wait(2)
        scale_scatter(PCHUNK - 1, 0)
        sca_wait(0)

        @pl.when(p_ + 1 < PASSES)
        def _():
            stage(p_ + 1)

        return 0

    lax.fori_loop(0, PASSES, one_pass, 0)
    plsc.subcore_barrier()
    for t in range(ROW_SLAB // CHUNK):
        pltpu.sync_copy(acc.at[pl.ds(r0 + t * CHUNK, CHUNK)],
                        out_hbm.at[c, pl.ds(r0 + t * CHUNK, CHUNK)])


_spmm_call = pl.kernel(
    _spmm_body,
    out_type=jax.ShapeDtypeStruct((NC, N, C), jnp.float32),
    mesh=_mesh,
    scratch_types=[
        pltpu.VMEM((PEDGE,), jnp.int32),
        pltpu.VMEM((PEDGE,), jnp.int32),
        pltpu.VMEM((PEDGE,), jnp.float32),
        pltpu.VMEM((CHUNK, C), jnp.float32),
        pltpu.VMEM((CHUNK, C), jnp.float32),
        pltpu.VMEM((CHUNK, C), jnp.float32),
        pltpu.VMEM_SHARED((N, C), jnp.float32),
    ] + [pltpu.SemaphoreType.DMA] * 7,
)


def _dinv_block(degp_ref):
    d = degp_ref[:, 0:1] + degp_ref[:, 1:2] + 1.0
    return lax.rsqrt(d)


def _prep_kernel(degp_ref, x_ref, g_ref):
    g_ref[...] = x_ref[...] * _dinv_block(degp_ref)


def _layer_kernel(degp_ref, sp_ref, g_ref, w_ref, b_ref, out_ref):
    dinv = _dinv_block(degp_ref)
    y = dinv * (sp_ref[0] + sp_ref[1] + g_ref[...])
    h = jnp.maximum(jnp.dot(y, w_ref[...],
                            preferred_element_type=jnp.float32) + b_ref[...],
                    0.0)
    out_ref[...] = dinv * h


def _final_kernel(degp_ref, sp_ref, g_ref, w_ref, b_ref, wfc_ref, bfc_ref,
                  out_ref, acc_ref):
    i = pl.program_id(0)

    @pl.when(i == 0)
    def _():
        acc_ref[...] = jnp.zeros_like(acc_ref)

    dinv = _dinv_block(degp_ref)
    y = dinv * (sp_ref[0] + sp_ref[1] + g_ref[...])
    h = jnp.maximum(jnp.dot(y, w_ref[...],
                            preferred_element_type=jnp.float32) + b_ref[...],
                    0.0)
    acc_ref[...] += jnp.sum(h, axis=0, keepdims=True)

    @pl.when(i == GRID - 1)
    def _():
        pooled = acc_ref[...] * (1.0 / N)
        out_ref[...] = jnp.dot(pooled, wfc_ref[...],
                               preferred_element_type=jnp.float32) + bfc_ref[...]


def _row_spec(rb, cols):
    return pl.BlockSpec((rb, cols), lambda i: (i, 0))


_degp_spec = pl.BlockSpec((RB, NC), lambda i: (i, 0))
_sp_spec = pl.BlockSpec((NC, RB, C), lambda i: (0, i, 0))
_full = lambda r, c: pl.BlockSpec((r, c), lambda i: (0, 0))

_prep_call = pl.pallas_call(
    _prep_kernel,
    grid=(GRID,),
    in_specs=[_degp_spec, _row_spec(RB, C)],
    out_specs=_row_spec(RB, C),
    out_shape=jax.ShapeDtypeStruct((N, C), jnp.float32),
)

_layer_call = pl.pallas_call(
    _layer_kernel,
    grid=(GRID,),
    in_specs=[_degp_spec, _sp_spec, _row_spec(RB, C),
              _full(C, C), _full(1, C)],
    out_specs=_row_spec(RB, C),
    out_shape=jax.ShapeDtypeStruct((N, C), jnp.float32),
)

_final_call = pl.pallas_call(
    _final_kernel,
    grid=(GRID,),
    in_specs=[_degp_spec, _sp_spec, _row_spec(RB, C),
              _full(C, C), _full(1, C), _full(C, 64), _full(1, 64)],
    out_specs=_full(1, 64),
    out_shape=jax.ShapeDtypeStruct((1, 64), jnp.float32),
    scratch_shapes=[pltpu.VMEM((1, C), jnp.float32)],
)


def kernel(x, edge_index, edge_weight, W1, b1, W2, b2, Wfc, bfc):
    src = edge_index[0].astype(jnp.int32)
    dst = edge_index[1].astype(jnp.int32)
    w = edge_weight.astype(jnp.float32)

    degp = _deg_call(dst, w)
    degp2 = degp.T

    g0 = _prep_call(degp2, x)
    s1 = _spmm_call(src, dst, w, g0)
    g1 = _layer_call(degp2, s1, g0, W1, b1.reshape(1, C))
    s2 = _spmm_call(src, dst, w, g1)
    out = _final_call(degp2, s2, g1, W2, b2.reshape(1, C),
                      Wfc, bfc.reshape(1, 64))
    return out


# speedup vs baseline: 1.0213x; 1.0213x over previous
import functools

import jax
import jax.numpy as jnp
from jax import lax
from jax.experimental import pallas as pl
from jax.experimental.pallas import tpu as pltpu
from jax.experimental.pallas import tpu_sc as plsc

N = 10000
E = 320000
C = 128
NC, NS = 2, 16
NW = NC * NS
EPW = E // NW
CHUNK = 80
NCHUNK = EPW // CHUNK
PASSES = 5
PCHUNK = NCHUNK // PASSES
PEDGE = PCHUNK * CHUNK
NPAD = 10240
DEG_SLAB = NPAD // NS
ROW_SLAB = 640
ROW_STRIDE = 624
RB = 2000
GRID = N // RB

_mesh = plsc.VectorSubcoreMesh(core_axis_name="c", subcore_axis_name="s")


def _zero_vmem_1d(ref, n):
    def body(i, _):
        ref[pl.ds(i * 16, 16)] = jnp.zeros((16,), jnp.float32)
        return 0
    lax.fori_loop(0, n // 16, body, 0)


def _deg_body(dst_hbm, w_hbm, deg_out, didx, wbuf, zbuf, acc, sem, sem2):
    c = lax.axis_index("c")
    s = lax.axis_index("s")
    wid = s * NC + c
    base = wid * EPW
    pltpu.async_copy(dst_hbm.at[pl.ds(base, EPW)], didx, sem)
    pltpu.async_copy(w_hbm.at[pl.ds(base, EPW)], wbuf, sem)
    _zero_vmem_1d(zbuf, DEG_SLAB)
    pltpu.sync_copy(zbuf, acc.at[pl.ds(s * DEG_SLAB, DEG_SLAB)])
    pltpu.make_async_copy(dst_hbm.at[pl.ds(base, EPW)], didx, sem).wait()
    pltpu.make_async_copy(w_hbm.at[pl.ds(base, EPW)], wbuf, sem).wait()
    plsc.subcore_barrier()

    def dsca(j, n, sm):
        pltpu.async_copy(wbuf.at[pl.ds(j * 128, n)],
                         acc.at[didx.at[pl.ds(j * 128, n)]], sm, add=True)

    def dsca_wait(n, sm):
        pltpu.make_async_copy(wbuf.at[pl.ds(0, n)],
                              acc.at[didx.at[pl.ds(0, n)]], sm).wait()

    dsca(0, 128, sem)
    dsca(1, 128, sem2)

    def dchunk(t, _):
        for p, sm in ((0, sem), (1, sem2)):
            j = 2 * t + p
            dsca_wait(128, sm)
            dsca(j + 2, 128, sm)
        return 0

    lax.fori_loop(0, 38, dchunk, 0)
    dsca_wait(128, sem)
    dsca_wait(128, sem2)
    dsca(78, 16, sem)
    dsca_wait(16, sem)
    plsc.subcore_barrier()
    pltpu.sync_copy(acc.at[pl.ds(s * DEG_SLAB, DEG_SLAB)],
                    deg_out.at[c, pl.ds(s * DEG_SLAB, DEG_SLAB)])


_deg_call = pl.kernel(
    _deg_body,
    out_type=jax.ShapeDtypeStruct((NC, NPAD), jnp.float32),
    mesh=_mesh,
    scratch_types=[
        pltpu.VMEM((EPW,), jnp.int32),
        pltpu.VMEM((EPW,), jnp.float32),
        pltpu.VMEM((DEG_SLAB,), jnp.float32),
        pltpu.VMEM_SHARED((NPAD,), jnp.float32),
        pltpu.SemaphoreType.DMA,
        pltpu.SemaphoreType.DMA,
    ],
)


def _scale_rows(rows, wbuf, j, q0=0, q1=CHUNK // 16):
    for q in range(q0, q1):
        wvec = wbuf[pl.ds(j * CHUNK + q * 16, 16)]

        def scale_row(r, _2, q=q, wvec=wvec):
            wspl = lax.gather(
                wvec, jnp.full((16, 1), r, jnp.int32),
                lax.GatherDimensionNumbers(
                    offset_dims=(), collapsed_slice_dims=(0,),
                    start_index_map=(0,)),
                (1,), mode=lax.GatherScatterMode.PROMISE_IN_BOUNDS)
            i = q * 16 + r
            for k in range(C // 16):
                rows[i, pl.ds(k * 16, 16)] = rows[i, pl.ds(k * 16, 16)] * wspl
            return 0

        lax.fori_loop(0, 16, scale_row, 0, unroll=4)


def _spmm_body(src_hbm, dst_hbm, w_hbm, g_hbm, out_hbm,
               sidx, didx, wbuf, rows0, rows1, rows2, acc,
               sems, semg0, semg1, semg2, semsc0, semsc1, semsc2):
    c = lax.axis_index("c")
    s = lax.axis_index("s")
    wid = s * NC + c
    rows = (rows0, rows1, rows2)
    semg = (semg0, semg1, semg2)
    semsc = (semsc0, semsc1, semsc2)

    def stage(p_):
        base = wid * EPW + p_ * PEDGE
        pltpu.async_copy(src_hbm.at[pl.ds(base, PEDGE)], sidx, sems)
        pltpu.async_copy(dst_hbm.at[pl.ds(base, PEDGE)], didx, sems)
        pltpu.async_copy(w_hbm.at[pl.ds(base, PEDGE)], wbuf, sems)

    def stage_wait(p_):
        base = wid * EPW + p_ * PEDGE
        pltpu.make_async_copy(src_hbm.at[pl.ds(base, PEDGE)], sidx, sems).wait()
        pltpu.make_async_copy(dst_hbm.at[pl.ds(base, PEDGE)], didx, sems).wait()
        pltpu.make_async_copy(w_hbm.at[pl.ds(base, PEDGE)], wbuf, sems).wait()

    stage(0)
    def zrow(i, _):
        def zcol(k, _2):
            rows0[i, pl.ds(k * 16, 16)] = jnp.zeros((16,), jnp.float32)
            return 0
        lax.fori_loop(0, C // 16, zcol, 0)
        return 0
    lax.fori_loop(0, CHUNK, zrow, 0)
    r0 = s * ROW_STRIDE
    for t in range(ROW_SLAB // CHUNK):
        pltpu.sync_copy(rows0, acc.at[pl.ds(r0 + t * CHUNK, CHUNK)])
    plsc.subcore_barrier()

    def gat(j, b):
        pltpu.async_copy(g_hbm.at[sidx.at[pl.ds(j * CHUNK, CHUNK)]],
                         rows[b], semg[b])

    def gat_wait(j, b):
        pltpu.make_async_copy(g_hbm.at[sidx.at[pl.ds(j * CHUNK, CHUNK)]],
                              rows[b], semg[b]).wait()

    H1 = 48

    def sca_part(j, b, r_off, nrows):
        pltpu.async_copy(rows[b].at[pl.ds(r_off, nrows)],
                         acc.at[didx.at[pl.ds(j * CHUNK + r_off, nrows)]],
                         semsc[b], add=True)

    def sca_wait(b):
        pltpu.make_async_copy(rows[b].at[pl.ds(0, H1)],
                              acc.at[didx.at[pl.ds(0, H1)]], semsc[b]).wait()
        pltpu.make_async_copy(rows[b].at[pl.ds(0, CHUNK - H1)],
                              acc.at[didx.at[pl.ds(0, CHUNK - H1)]],
                              semsc[b]).wait()

    def scale_scatter(j, b):
        _scale_rows(rows[b], wbuf, j, 0, H1 // 16)
        sca_part(j, b, 0, H1)
        _scale_rows(rows[b], wbuf, j, H1 // 16, CHUNK // 16)
        sca_part(j, b, H1, CHUNK - H1)

    def one_pass(p_, _):
        stage_wait(p_)
        gat(0, 0)
        gat(1, 1)
        gat_wait(0, 0)
        gat(2, 2)
        scale_scatter(0, 0)
        gat_wait(1, 1)
        sca_wait(0)
        gat(3, 0)
        scale_scatter(1, 1)

        def triple(t, _):
            for off, b, b2 in ((2, 2, 1), (3, 0, 2), (4, 1, 0)):
                j = 3 * t + off
                gat_wait(j, b)
                sca_wait(b2)
                gat(j + 2, b2)
                scale_scatter(j, b)
            return 0

        lax.fori_loop(0, (PCHUNK - 4) // 3, triple, 0)
        gat_wait(PCHUNK - 2, 2)
        sca_wait(1)
        scale_scatter(PCHUNK - 2, 2)
        gat_wait(PCHUNK - 1, 0)
        sca_wait(2)
        scale_scatter(PCHUNK - 1, 0)
        sca_wait(0)

        @pl.when(p_ + 1 < PASSES)
        def _():
            stage(p_ + 1)

        return 0

    lax.fori_loop(0, PASSES, one_pass, 0)
    plsc.subcore_barrier()
    for t in range(ROW_SLAB // CHUNK):
        pltpu.sync_copy(acc.at[pl.ds(r0 + t * CHUNK, CHUNK)],
                        out_hbm.at[c, pl.ds(r0 + t * CHUNK, CHUNK)])


_spmm_call = pl.kernel(
    _spmm_body,
    out_type=jax.ShapeDtypeStruct((NC, N, C), jnp.float32),
    mesh=_mesh,
    scratch_types=[
        pltpu.VMEM((PEDGE,), jnp.int32),
        pltpu.VMEM((PEDGE,), jnp.int32),
        pltpu.VMEM((PEDGE,), jnp.float32),
        pltpu.VMEM((CHUNK, C), jnp.float32),
        pltpu.VMEM((CHUNK, C), jnp.float32),
        pltpu.VMEM((CHUNK, C), jnp.float32),
        pltpu.VMEM_SHARED((N, C), jnp.float32),
    ] + [pltpu.SemaphoreType.DMA] * 7,
)


def _dinv_block(degp_ref):
    d = degp_ref[:, 0:1] + degp_ref[:, 1:2] + 1.0
    return lax.rsqrt(d)


def _prep_kernel(degp_ref, x_ref, g_ref):
    g_ref[...] = x_ref[...] * _dinv_block(degp_ref)


def _layer_kernel(degp_ref, sp_ref, g_ref, w_ref, b_ref, out_ref):
    dinv = _dinv_block(degp_ref)
    y = dinv * (sp_ref[0] + sp_ref[1] + g_ref[...])
    h = jnp.maximum(jnp.dot(y, w_ref[...],
                            preferred_element_type=jnp.float32) + b_ref[...],
                    0.0)
    out_ref[...] = dinv * h


def _final_kernel(degp_ref, sp_ref, g_ref, w_ref, b_ref, wfc_ref, bfc_ref,
                  out_ref, acc_ref):
    i = pl.program_id(0)

    @pl.when(i == 0)
    def _():
        acc_ref[...] = jnp.zeros_like(acc_ref)

    dinv = _dinv_block(degp_ref)
    y = dinv * (sp_ref[0] + sp_ref[1] + g_ref[...])
    h = jnp.maximum(jnp.dot(y, w_ref[...],
                            preferred_element_type=jnp.float32) + b_ref[...],
                    0.0)
    acc_ref[...] += jnp.sum(h, axis=0, keepdims=True)

    @pl.when(i == GRID - 1)
    def _():
        pooled = acc_ref[...] * (1.0 / N)
        out_ref[...] = jnp.dot(pooled, wfc_ref[...],
                               preferred_element_type=jnp.float32) + bfc_ref[...]


def _row_spec(rb, cols):
    return pl.BlockSpec((rb, cols), lambda i: (i, 0))


_degp_spec = pl.BlockSpec((RB, NC), lambda i: (i, 0))
_sp_spec = pl.BlockSpec((NC, RB, C), lambda i: (0, i, 0))
_full = lambda r, c: pl.BlockSpec((r, c), lambda i: (0, 0))

_prep_call = pl.pallas_call(
    _prep_kernel,
    grid=(GRID,),
    in_specs=[_degp_spec, _row_spec(RB, C)],
    out_specs=_row_spec(RB, C),
    out_shape=jax.ShapeDtypeStruct((N, C), jnp.float32),
)

_layer_call = pl.pallas_call(
    _layer_kernel,
    grid=(GRID,),
    in_specs=[_degp_spec, _sp_spec, _row_spec(RB, C),
              _full(C, C), _full(1, C)],
    out_specs=_row_spec(RB, C),
    out_shape=jax.ShapeDtypeStruct((N, C), jnp.float32),
)

_final_call = pl.pallas_call(
    _final_kernel,
    grid=(GRID,),
    in_specs=[_degp_spec, _sp_spec, _row_spec(RB, C),
              _full(C, C), _full(1, C), _full(C, 64), _full(1, 64)],
    out_specs=_full(1, 64),
    out_shape=jax.ShapeDtypeStruct((1, 64), jnp.float32),
    scratch_shapes=[pltpu.VMEM((1, C), jnp.float32)],
)


def kernel(x, edge_index, edge_weight, W1, b1, W2, b2, Wfc, bfc):
    src = edge_index[0].astype(jnp.int32)
    dst = edge_index[1].astype(jnp.int32)
    w = edge_weight.astype(jnp.float32)

    degp = _deg_call(dst, w)
    degp2 = degp.T

    g0 = _prep_call(degp2, x)
    s1 = _spmm_call(src, dst, w, g0)
    g1 = _layer_call(degp2, s1, g0, W1, b1.reshape(1, C))
    s2 = _spmm_call(src, dst, w, g1)
    out = _final_call(degp2, s2, g1, W2, b2.reshape(1, C),
                      Wfc, bfc.reshape(1, 64))
    return out


# speedup vs baseline: 1.0213x; 1.0001x over previous
import jax
import jax.numpy as jnp
from jax import lax
from jax.experimental import pallas as pl
from jax.experimental.pallas import tpu as pltpu
from jax.experimental.pallas import tpu_sc as plsc

N = 10000
E = 320000
C = 128
NC, NS = 2, 16
NW = NC * NS
EPW = E // NW
CHUNK = 80
NCHUNK = EPW // CHUNK
PASSES = 5
PCHUNK = NCHUNK // PASSES
PEDGE = PCHUNK * CHUNK
NPAD = 10240
DEG_SLAB = NPAD // NS
ROW_SLAB = 640
ROW_STRIDE = 624
RB = 2000
GRID = N // RB

_mesh = plsc.VectorSubcoreMesh(core_axis_name="c", subcore_axis_name="s")


def _zero_vmem_1d(ref, n):
    def body(i, _):
        ref[pl.ds(i * 16, 16)] = jnp.zeros((16,), jnp.float32)
        return 0
    lax.fori_loop(0, n // 16, body, 0)


def _deg_body(dst_hbm, w_hbm, deg_out, didx, wbuf, zbuf, acc, sem, sem2):
    c = lax.axis_index("c")
    s = lax.axis_index("s")
    wid = s * NC + c
    base = wid * EPW
    pltpu.async_copy(dst_hbm.at[pl.ds(base, EPW)], didx, sem)
    pltpu.async_copy(w_hbm.at[pl.ds(base, EPW)], wbuf, sem)
    _zero_vmem_1d(zbuf, DEG_SLAB)
    pltpu.sync_copy(zbuf, acc.at[pl.ds(s * DEG_SLAB, DEG_SLAB)])
    pltpu.make_async_copy(dst_hbm.at[pl.ds(base, EPW)], didx, sem).wait()
    pltpu.make_async_copy(w_hbm.at[pl.ds(base, EPW)], wbuf, sem).wait()
    plsc.subcore_barrier()

    def dsca(j, n, sm):
        pltpu.async_copy(wbuf.at[pl.ds(j * 128, n)],
                         acc.at[didx.at[pl.ds(j * 128, n)]], sm, add=True)

    def dsca_wait(n, sm):
        pltpu.make_async_copy(wbuf.at[pl.ds(0, n)],
                              acc.at[didx.at[pl.ds(0, n)]], sm).wait()

    dsca(0, 128, sem)
    dsca(1, 128, sem2)

    def dchunk(t, _):
        for p, sm in ((0, sem), (1, sem2)):
            j = 2 * t + p
            dsca_wait(128, sm)
            dsca(j + 2, 128, sm)
        return 0

    lax.fori_loop(0, 38, dchunk, 0)
    dsca_wait(128, sem)
    dsca_wait(128, sem2)
    dsca(78, 16, sem)
    dsca_wait(16, sem)
    plsc.subcore_barrier()
    pltpu.sync_copy(acc.at[pl.ds(s * DEG_SLAB, DEG_SLAB)],
                    deg_out.at[c, pl.ds(s * DEG_SLAB, DEG_SLAB)])


_deg_call = pl.kernel(
    _deg_body,
    out_type=jax.ShapeDtypeStruct((NC, NPAD), jnp.float32),
    mesh=_mesh,
    scratch_types=[
        pltpu.VMEM((EPW,), jnp.int32),
        pltpu.VMEM((EPW,), jnp.float32),
        pltpu.VMEM((DEG_SLAB,), jnp.float32),
        pltpu.VMEM_SHARED((NPAD,), jnp.float32),
        pltpu.SemaphoreType.DMA,
        pltpu.SemaphoreType.DMA,
    ],
)


def _scale_rows(rows, wbuf, j, q0=0, q1=CHUNK // 16):
    for q in range(q0, q1):
        wvec = wbuf[pl.ds(j * CHUNK + q * 16, 16)]

        def scale_row(r, _2, q=q, wvec=wvec):
            wspl = lax.gather(
                wvec, jnp.full((16, 1), r, jnp.int32),
                lax.GatherDimensionNumbers(
                    offset_dims=(), collapsed_slice_dims=(0,),
                    start_index_map=(0,)),
                (1,), mode=lax.GatherScatterMode.PROMISE_IN_BOUNDS)
            i = q * 16 + r
            for k in range(C // 16):
                rows[i, pl.ds(k * 16, 16)] = rows[i, pl.ds(k * 16, 16)] * wspl
            return 0

        lax.fori_loop(0, 16, scale_row, 0, unroll=4)


def _spmm_body(src_hbm, dst_hbm, w_hbm, g_hbm, out_hbm,
               sidx, didx, wbuf, rows0, rows1, rows2, acc,
               sems, semg0, semg1, semg2, semsc0, semsc1, semsc2):
    c = lax.axis_index("c")
    s = lax.axis_index("s")
    wid = s * NC + c
    rows = (rows0, rows1, rows2)
    semg = (semg0, semg1, semg2)
    semsc = (semsc0, semsc1, semsc2)

    def stage(p_):
        base = wid * EPW + p_ * PEDGE
        pltpu.async_copy(src_hbm.at[pl.ds(base, PEDGE)], sidx, sems)
        pltpu.async_copy(dst_hbm.at[pl.ds(base, PEDGE)], didx, sems)
        pltpu.async_copy(w_hbm.at[pl.ds(base, PEDGE)], wbuf, sems)

    def stage_wait(p_):
        base = wid * EPW + p_ * PEDGE
        pltpu.make_async_copy(src_hbm.at[pl.ds(base, PEDGE)], sidx, sems).wait()
        pltpu.make_async_copy(dst_hbm.at[pl.ds(base, PEDGE)], didx, sems).wait()
        pltpu.make_async_copy(w_hbm.at[pl.ds(base, PEDGE)], wbuf, sems).wait()

    stage(0)
    def zrow(i, _):
        def zcol(k, _2):
            rows0[i, pl.ds(k * 16, 16)] = jnp.zeros((16,), jnp.float32)
            return 0
        lax.fori_loop(0, C // 16, zcol, 0)
        return 0
    lax.fori_loop(0, CHUNK, zrow, 0)
    r0 = s * ROW_STRIDE
    for t in range(ROW_SLAB // CHUNK):
        pltpu.sync_copy(rows0, acc.at[pl.ds(r0 + t * CHUNK, CHUNK)])
    plsc.subcore_barrier()

    def gat(j, b):
        pltpu.async_copy(g_hbm.at[sidx.at[pl.ds(j * CHUNK, CHUNK)]],
                         rows[b], semg[b])

    def gat_wait(j, b):
        pltpu.make_async_copy(g_hbm.at[sidx.at[pl.ds(j * CHUNK, CHUNK)]],
                              rows[b], semg[b]).wait()

    H1 = 48

    def sca_part(j, b, r_off, nrows):
        pltpu.async_copy(rows[b].at[pl.ds(r_off, nrows)],
                         acc.at[didx.at[pl.ds(j * CHUNK + r_off, nrows)]],
                         semsc[b], add=True)

    def sca_wait(b):
        pltpu.make_async_copy(rows[b].at[pl.ds(0, H1)],
                              acc.at[didx.at[pl.ds(0, H1)]], semsc[b]).wait()
        pltpu.make_async_copy(rows[b].at[pl.ds(0, CHUNK - H1)],
                              acc.at[didx.at[pl.ds(0, CHUNK - H1)]],
                              semsc[b]).wait()

    def scale_scatter(j, b):
        _scale_rows(rows[b], wbuf, j, 0, H1 // 16)
        sca_part(j, b, 0, H1)
        _scale_rows(rows[b], wbuf, j, H1 // 16, CHUNK // 16)
        sca_part(j, b, H1, CHUNK - H1)

    def one_pass(p_, _):
        stage_wait(p_)
        gat(0, 0)
        gat(1, 1)
        gat_wait(0, 0)
        gat(2, 2)
        scale_scatter(0, 0)
        gat_wait(1, 1)
        sca_wait(0)
        gat(3, 0)
        scale_scatter(1, 1)

        def triple(t, _):
            for off, b, b2 in ((2, 2, 1), (3, 0, 2), (4, 1, 0)):
                j = 3 * t + off
                gat_wait(j, b)
                sca_wait(b2)
                gat(j + 2, b2)
                scale_scatter(j, b)
            return 0

        lax.fori_loop(0, (PCHUNK - 4) // 3, triple, 0)
        gat_wait(PCHUNK - 2, 2)
        sca_wait(1)
        scale_scatter(PCHUNK - 2, 2)
        gat_wait(PCHUNK - 1, 0)
        sca_wait(2)
        scale_scatter(PCHUNK - 1, 0)
        sca_wait(0)

        @pl.when(p_ + 1 < PASSES)
        def _():
            stage(p_ + 1)

        return 0

    lax.fori_loop(0, PASSES, one_pass, 0)
    plsc.subcore_barrier()
    for t in range(ROW_SLAB // CHUNK):
        pltpu.sync_copy(acc.at[pl.ds(r0 + t * CHUNK, CHUNK)],
                        out_hbm.at[c, pl.ds(r0 + t * CHUNK, CHUNK)])


_spmm_call = pl.kernel(
    _spmm_body,
    out_type=jax.ShapeDtypeStruct((NC, N, C), jnp.float32),
    mesh=_mesh,
    scratch_types=[
        pltpu.VMEM((PEDGE,), jnp.int32),
        pltpu.VMEM((PEDGE,), jnp.int32),
        pltpu.VMEM((PEDGE,), jnp.float32),
        pltpu.VMEM((CHUNK, C), jnp.float32),
        pltpu.VMEM((CHUNK, C), jnp.float32),
        pltpu.VMEM((CHUNK, C), jnp.float32),
        pltpu.VMEM_SHARED((N, C), jnp.float32),
    ] + [pltpu.SemaphoreType.DMA] * 7,
)


def _dinv_block(degp_ref):
    d = degp_ref[:, 0:1] + degp_ref[:, 1:2] + 1.0
    return lax.rsqrt(d)


def _prep_kernel(degp_ref, x_ref, g_ref):
    g_ref[...] = x_ref[...] * _dinv_block(degp_ref)


def _layer_kernel(degp_ref, sp_ref, g_ref, w_ref, b_ref, out_ref):
    dinv = _dinv_block(degp_ref)
    y = dinv * (sp_ref[0] + sp_ref[1] + g_ref[...])
    h = jnp.maximum(jnp.dot(y, w_ref[...],
                            preferred_element_type=jnp.float32) + b_ref[...],
                    0.0)
    out_ref[...] = dinv * h


def _final_kernel(degp_ref, sp_ref, g_ref, w_ref, b_ref, wfc_ref, bfc_ref,
                  out_ref, acc_ref):
    i = pl.program_id(0)

    @pl.when(i == 0)
    def _():
        acc_ref[...] = jnp.zeros_like(acc_ref)

    dinv = _dinv_block(degp_ref)
    y = dinv * (sp_ref[0] + sp_ref[1] + g_ref[...])
    h = jnp.maximum(jnp.dot(y, w_ref[...],
                            preferred_element_type=jnp.float32) + b_ref[...],
                    0.0)
    acc_ref[...] += jnp.sum(h, axis=0, keepdims=True)

    @pl.when(i == GRID - 1)
    def _():
        pooled = acc_ref[...] * (1.0 / N)
        out_ref[...] = jnp.dot(pooled, wfc_ref[...],
                               preferred_element_type=jnp.float32) + bfc_ref[...]


def _row_spec(rb, cols):
    return pl.BlockSpec((rb, cols), lambda i: (i, 0))


_degp_spec = pl.BlockSpec((RB, NC), lambda i: (i, 0))
_sp_spec = pl.BlockSpec((NC, RB, C), lambda i: (0, i, 0))
_full = lambda r, c: pl.BlockSpec((r, c), lambda i: (0, 0))

_prep_call = pl.pallas_call(
    _prep_kernel,
    grid=(GRID,),
    in_specs=[_degp_spec, _row_spec(RB, C)],
    out_specs=_row_spec(RB, C),
    out_shape=jax.ShapeDtypeStruct((N, C), jnp.float32),
)

_layer_call = pl.pallas_call(
    _layer_kernel,
    grid=(GRID,),
    in_specs=[_degp_spec, _sp_spec, _row_spec(RB, C),
              _full(C, C), _full(1, C)],
    out_specs=_row_spec(RB, C),
    out_shape=jax.ShapeDtypeStruct((N, C), jnp.float32),
)

_final_call = pl.pallas_call(
    _final_kernel,
    grid=(GRID,),
    in_specs=[_degp_spec, _sp_spec, _row_spec(RB, C),
              _full(C, C), _full(1, C), _full(C, 64), _full(1, 64)],
    out_specs=_full(1, 64),
    out_shape=jax.ShapeDtypeStruct((1, 64), jnp.float32),
    scratch_shapes=[pltpu.VMEM((1, C), jnp.float32)],
)


def kernel(x, edge_index, edge_weight, W1, b1, W2, b2, Wfc, bfc):
    src = edge_index[0].astype(jnp.int32)
    dst = edge_index[1].astype(jnp.int32)
    w = edge_weight.astype(jnp.float32)

    degp = _deg_call(dst, w)
    degp2 = degp.T

    g0 = _prep_call(degp2, x)
    s1 = _spmm_call(src, dst, w, g0)
    g1 = _layer_call(degp2, s1, g0, W1, b1.reshape(1, C))
    s2 = _spmm_call(src, dst, w, g1)
    out = _final_call(degp2, s2, g1, W2, b2.reshape(1, C),
                      Wfc, bfc.reshape(1, 64))
    return out
